# Initial kernel scaffold; baseline (speedup 1.0000x reference)
import jax
import jax.numpy as jnp
from jax.experimental import pallas as pl


def kernel(scores, deltas, anchors):
    raise NotImplementedError("write your pallas kernel here")


# speedup vs baseline: 6.7771x; 6.7771x over previous
import functools

import jax
import jax.numpy as jnp
from jax import lax
from jax.experimental import pallas as pl

_PROPOSALS = 1000
_NMS_THR = 0.7
_K = 6000
_STD = (0.1, 0.1, 0.2, 0.2)

_ROWS = 160
_LANES = 128
_NPAD = _ROWS * _LANES
_NEG = -3.0e38
_SUPPRESSED = -1.0e10


def _nms_body(fg_ref, d_ref, a_ref, out_ref):
    fg = fg_ref[0]
    ibits = lax.bitcast_convert_type(fg, jnp.int32)
    key = ibits ^ ((ibits >> 31) & jnp.int32(0x7FFFFFFF))

    ridx = lax.broadcasted_iota(jnp.int32, (_ROWS, _LANES), 0)
    cidx = lax.broadcasted_iota(jnp.int32, (_ROWS, _LANES), 1)
    gidx = ridx * _LANES + cidx

    kf = jnp.float32(_K)

    def _avg_floor(a, b):
        return (a >> 1) + (b >> 1) + (a & b & 1)

    def _search_step(_, carry):
        lo, hi = carry
        mid = _avg_floor(lo, hi)
        cnt = jnp.sum((key >= mid).astype(jnp.float32))
        ge = cnt >= kf
        new_lo = jnp.where(ge, mid, lo)
        new_hi = jnp.where(ge, hi, mid)
        prog = hi > lo + 1
        return (jnp.where(prog, new_lo, lo), jnp.where(prog, new_hi, hi))

    lo0 = jnp.int32(-2147483647 - 1)
    hi0 = jnp.int32(2147483647)
    t, _ = lax.fori_loop(0, 33, _search_step, (lo0, hi0))

    n_gt = jnp.sum((key > t).astype(jnp.float32))
    need_ties = kf - n_gt
    is_tie = key == t

    def _jsearch(_, carry):
        jlo, jhi = carry
        jmid = (jlo + jhi) >> 1
        cnt = jnp.sum((is_tie & (gidx < jmid)).astype(jnp.float32))
        ge = cnt >= need_ties
        new_hi = jnp.where(ge, jmid, jhi)
        new_lo = jnp.where(ge, jlo, jmid + 1)
        prog = jhi > jlo
        return (jnp.where(prog, new_lo, jlo), jnp.where(prog, new_hi, jhi))

    _, jcut = lax.fori_loop(0, 16, _jsearch, (jnp.int32(0), jnp.int32(_NPAD)))

    cand = (key > t) | (is_tie & (gidx < jcut))

    d0 = d_ref[0, 0] * jnp.float32(_STD[0])
    d1 = d_ref[0, 1] * jnp.float32(_STD[1])
    d2 = d_ref[0, 2] * jnp.float32(_STD[2])
    d3 = d_ref[0, 3] * jnp.float32(_STD[3])
    a0 = a_ref[0, 0]
    a1 = a_ref[0, 1]
    a2 = a_ref[0, 2]
    a3 = a_ref[0, 3]
    height = a2 - a0
    width = a3 - a1
    cy = a0 + 0.5 * height + d0 * height
    cx = a1 + 0.5 * width + d1 * width
    height = height * jnp.exp(d2)
    width = width * jnp.exp(d3)
    y1 = jnp.clip(cy - 0.5 * height, 0.0, 1.0)
    x1 = jnp.clip(cx - 0.5 * width, 0.0, 1.0)
    y2 = jnp.clip((cy - 0.5 * height) + height, 0.0, 1.0)
    x2 = jnp.clip((cx - 0.5 * width) + width, 0.0, 1.0)
    areas = (y2 - y1) * (x2 - x1)

    sw0 = jnp.where(cand, fg, jnp.float32(_NEG))

    io4 = lax.broadcasted_iota(jnp.int32, (1, 4), 1)

    def _step(p, sw):
        m = jnp.max(sw)
        valid = m > -1.0e9
        eq = sw == m
        idxsel = jnp.min(jnp.where(eq, gidx, jnp.int32(_NPAD)))
        sel = (eq & (gidx == idxsel)).astype(jnp.float32)
        by1 = jnp.sum(y1 * sel)
        bx1 = jnp.sum(x1 * sel)
        by2 = jnp.sum(y2 * sel)
        bx2 = jnp.sum(x2 * sel)
        barea = (by2 - by1) * (bx2 - bx1)
        yy1 = jnp.maximum(by1, y1)
        xx1 = jnp.maximum(bx1, x1)
        yy2 = jnp.minimum(by2, y2)
        xx2 = jnp.minimum(bx2, x2)
        inter = jnp.maximum(yy2 - yy1, 0.0) * jnp.maximum(xx2 - xx1, 0.0)
        iou = inter / (barea + areas - inter + 1e-8)
        suppress = (iou > _NMS_THR) | (gidx == idxsel)
        sw = jnp.where(suppress & valid, jnp.float32(_SUPPRESSED), sw)
        sw = jnp.where(cand, sw, jnp.float32(_NEG))
        vf = jnp.where(valid, 1.0, 0.0).astype(jnp.float32)
        row = jnp.where(io4 == 0, by1,
                        jnp.where(io4 == 1, bx1,
                                  jnp.where(io4 == 2, by2, bx2))) * vf
        out_ref[0, pl.ds(p, 1), :] = row
        return sw

    lax.fori_loop(0, _PROPOSALS, _step, sw0)


def kernel(scores, deltas, anchors):
    B, N, _ = scores.shape
    fg = scores[:, :, 1]
    pad = _NPAD - N
    fg = jnp.pad(fg, ((0, 0), (0, pad)), constant_values=_NEG)
    fg = fg.reshape(B, _ROWS, _LANES)
    d = jnp.moveaxis(deltas, 2, 1)
    a = jnp.moveaxis(anchors, 2, 1)
    d = jnp.pad(d, ((0, 0), (0, 0), (0, pad)))
    a = jnp.pad(a, ((0, 0), (0, 0), (0, pad)))
    d = d.reshape(B, 4, _ROWS, _LANES)
    a = a.reshape(B, 4, _ROWS, _LANES)

    out = pl.pallas_call(
        _nms_body,
        grid=(B,),
        in_specs=[
            pl.BlockSpec((1, _ROWS, _LANES), lambda b: (b, 0, 0)),
            pl.BlockSpec((1, 4, _ROWS, _LANES), lambda b: (b, 0, 0, 0)),
            pl.BlockSpec((1, 4, _ROWS, _LANES), lambda b: (b, 0, 0, 0)),
        ],
        out_specs=pl.BlockSpec((1, _PROPOSALS, 4), lambda b: (b, 0, 0)),
        out_shape=jax.ShapeDtypeStruct((B, _PROPOSALS, 4), jnp.float32),
    )(fg, d, a)
    return out


# speedup vs baseline: 8.3176x; 1.2273x over previous
import functools

import jax
import jax.numpy as jnp
from jax import lax
from jax.experimental import pallas as pl
from jax.experimental.pallas import tpu as pltpu
from jax.experimental.pallas import tpu_sc as plsc

_PROPOSALS = 1000
_NMS_THR = 0.7
_K = 6000
_STD = (0.1, 0.1, 0.2, 0.2)

_ROWS = 160
_LANES = 128
_NPAD = _ROWS * _LANES
_NEG = -3.0e38
_SUPPRESSED = -1.0e10

_CROWS = 48
_CN = _CROWS * _LANES
_COUT = 6400
_TRASH = 6144
_RW = 16

_NW = 32
_B = 2


def _prep_body(fg_ref, d_ref, a_ref, planes_ref, sidx_ref):
    b = pl.program_id(0)
    fg = fg_ref[0]
    ibits = lax.bitcast_convert_type(fg, jnp.int32)
    key = ibits ^ ((ibits >> 31) & jnp.int32(0x7FFFFFFF))

    ridx = lax.broadcasted_iota(jnp.int32, (_ROWS, _LANES), 0)
    cidx = lax.broadcasted_iota(jnp.int32, (_ROWS, _LANES), 1)
    gidx = ridx * _LANES + cidx

    kf = jnp.float32(_K)

    def _avg_floor(a, b2):
        return (a >> 1) + (b2 >> 1) + (a & b2 & 1)

    def _search_step(_, carry):
        lo, hi = carry
        mid = _avg_floor(lo, hi)
        cnt = jnp.sum((key >= mid).astype(jnp.float32))
        ge = cnt >= kf
        new_lo = jnp.where(ge, mid, lo)
        new_hi = jnp.where(ge, hi, mid)
        prog = hi > lo + 1
        return (jnp.where(prog, new_lo, lo), jnp.where(prog, new_hi, hi))

    lo0 = jnp.int32(-2147483647 - 1)
    hi0 = jnp.int32(2147483647)
    t, _ = lax.fori_loop(0, 33, _search_step, (lo0, hi0))

    n_gt = jnp.sum((key > t).astype(jnp.float32))
    need_ties = kf - n_gt
    is_tie = key == t

    def _jsearch(_, carry):
        jlo, jhi = carry
        jmid = (jlo + jhi) >> 1
        cnt = jnp.sum((is_tie & (gidx < jmid)).astype(jnp.float32))
        ge = cnt >= need_ties
        new_hi = jnp.where(ge, jmid, jhi)
        new_lo = jnp.where(ge, jlo, jmid + 1)
        prog = jhi > jlo
        return (jnp.where(prog, new_lo, jlo), jnp.where(prog, new_hi, jhi))

    _, jcut = lax.fori_loop(0, 16, _jsearch, (jnp.int32(0), jnp.int32(_NPAD)))

    cand = (key > t) | (is_tie & (gidx < jcut))

    d0 = d_ref[0, 0] * jnp.float32(_STD[0])
    d1 = d_ref[0, 1] * jnp.float32(_STD[1])
    d2 = d_ref[0, 2] * jnp.float32(_STD[2])
    d3 = d_ref[0, 3] * jnp.float32(_STD[3])
    a0 = a_ref[0, 0]
    a1 = a_ref[0, 1]
    a2 = a_ref[0, 2]
    a3 = a_ref[0, 3]
    height = a2 - a0
    width = a3 - a1
    cy = a0 + 0.5 * height + d0 * height
    cx = a1 + 0.5 * width + d1 * width
    height = height * jnp.exp(d2)
    width = width * jnp.exp(d3)
    y1 = jnp.clip(cy - 0.5 * height, 0.0, 1.0)
    x1 = jnp.clip(cx - 0.5 * width, 0.0, 1.0)
    y2 = jnp.clip((cy - 0.5 * height) + height, 0.0, 1.0)
    x2 = jnp.clip((cx - 0.5 * width) + width, 0.0, 1.0)
    areas = (y2 - y1) * (x2 - x1)

    candf = cand.astype(jnp.float32)
    lane_i = lax.broadcasted_iota(jnp.int32, (_LANES, _LANES), 0)
    lane_j = lax.broadcasted_iota(jnp.int32, (_LANES, _LANES), 1)
    upper_strict = (lane_i < lane_j).astype(jnp.float32)
    lane_excl = lax.dot_general(candf, upper_strict,
                                (((1,), (0,)), ((), ())),
                                preferred_element_type=jnp.float32)
    row_i = lax.broadcasted_iota(jnp.int32, (_ROWS, _ROWS), 0)
    row_j = lax.broadcasted_iota(jnp.int32, (_ROWS, _ROWS), 1)
    lower_strict = (row_i > row_j).astype(jnp.float32)
    rowtot = jnp.sum(candf, axis=1, keepdims=True)
    row_excl = lax.dot_general(lower_strict, rowtot,
                               (((1,), (0,)), ((), ())),
                               preferred_element_type=jnp.float32)
    off = (row_excl + lane_excl).astype(jnp.int32)

    trash = jnp.int32(_TRASH) + (gidx & jnp.int32(127))
    slot = jnp.where(cand, off, trash) + b * jnp.int32(_COUT)
    sidx_ref[0] = slot

    planes_ref[0, 0] = jnp.where(cand, fg, jnp.float32(_NEG))
    planes_ref[0, 1] = y1
    planes_ref[0, 2] = x1
    planes_ref[0, 3] = y2
    planes_ref[0, 4] = x2
    planes_ref[0, 5] = areas


def _sc_compact(rows_hbm, sidx_hbm, out_hbm, idx_v, rows_v, sem):
    w = lax.axis_index("s") * 2 + lax.axis_index("c")
    pltpu.sync_copy(sidx_hbm.at[w], idx_v)
    pltpu.sync_copy(rows_hbm.at[w], rows_v)
    ch = idx_v.shape[0]
    cps = [pltpu.async_copy(rows_v.at[c], out_hbm.at[idx_v.at[c]], sem)
           for c in range(ch)]
    for cp in cps:
        cp.wait()


def _nms_body(p_ref, out_ref):
    gidx = (lax.broadcasted_iota(jnp.int32, (_CROWS, _LANES), 0) * _LANES
            + lax.broadcasted_iota(jnp.int32, (_CROWS, _LANES), 1))
    cand = gidx < _K
    io4 = lax.broadcasted_iota(jnp.int32, (1, 4), 1)

    def load(b):
        s = jnp.where(cand, p_ref[b, 0], jnp.float32(_NEG))
        return (s, p_ref[b, 1], p_ref[b, 2], p_ref[b, 3], p_ref[b, 4],
                p_ref[b, 5])

    st = [load(b) for b in range(_B)]

    def _step(p, sws):
        new_sws = []
        for b in range(_B):
            sw = sws[b]
            _, y1, x1, y2, x2, areas = st[b]
            m = jnp.max(sw)
            valid = m > -1.0e9
            eq = sw == m
            idxsel = jnp.min(jnp.where(eq, gidx, jnp.int32(_CN)))
            sel = (eq & (gidx == idxsel)).astype(jnp.float32)
            by1 = jnp.sum(y1 * sel)
            bx1 = jnp.sum(x1 * sel)
            by2 = jnp.sum(y2 * sel)
            bx2 = jnp.sum(x2 * sel)
            barea = (by2 - by1) * (bx2 - bx1)
            yy1 = jnp.maximum(by1, y1)
            xx1 = jnp.maximum(bx1, x1)
            yy2 = jnp.minimum(by2, y2)
            xx2 = jnp.minimum(bx2, x2)
            inter = jnp.maximum(yy2 - yy1, 0.0) * jnp.maximum(xx2 - xx1, 0.0)
            iou = inter / (barea + areas - inter + 1e-8)
            suppress = (iou > _NMS_THR) | (gidx == idxsel)
            sw = jnp.where(suppress & valid, jnp.float32(_SUPPRESSED), sw)
            sw = jnp.where(cand, sw, jnp.float32(_NEG))
            vf = jnp.where(valid, 1.0, 0.0).astype(jnp.float32)
            row = jnp.where(io4 == 0, by1,
                            jnp.where(io4 == 1, bx1,
                                      jnp.where(io4 == 2, by2, bx2))) * vf
            out_ref[b, pl.ds(p, 1), :] = row
            new_sws.append(sw)
        return tuple(new_sws)

    lax.fori_loop(0, _PROPOSALS, _step, tuple(s[0] for s in st))


def kernel(scores, deltas, anchors):
    B, N, _ = scores.shape
    fg = scores[:, :, 1]
    pad = _NPAD - N
    fg = jnp.pad(fg, ((0, 0), (0, pad)), constant_values=_NEG)
    fg = fg.reshape(B, _ROWS, _LANES)
    d = jnp.moveaxis(deltas, 2, 1)
    a = jnp.moveaxis(anchors, 2, 1)
    d = jnp.pad(d, ((0, 0), (0, 0), (0, pad)))
    a = jnp.pad(a, ((0, 0), (0, 0), (0, pad)))
    d = d.reshape(B, 4, _ROWS, _LANES)
    a = a.reshape(B, 4, _ROWS, _LANES)

    planes, sidx = pl.pallas_call(
        _prep_body,
        grid=(B,),
        in_specs=[
            pl.BlockSpec((1, _ROWS, _LANES), lambda b: (b, 0, 0)),
            pl.BlockSpec((1, 4, _ROWS, _LANES), lambda b: (b, 0, 0, 0)),
            pl.BlockSpec((1, 4, _ROWS, _LANES), lambda b: (b, 0, 0, 0)),
        ],
        out_specs=[
            pl.BlockSpec((1, 6, _ROWS, _LANES), lambda b: (b, 0, 0, 0)),
            pl.BlockSpec((1, _ROWS, _LANES), lambda b: (b, 0, 0)),
        ],
        out_shape=[
            jax.ShapeDtypeStruct((B, 6, _ROWS, _LANES), jnp.float32),
            jax.ShapeDtypeStruct((B, _ROWS, _LANES), jnp.int32),
        ],
    )(fg, d, a)

    rows = jnp.moveaxis(planes.reshape(B, 6, _NPAD), 1, 2)
    rows = jnp.pad(rows, ((0, 0), (0, 0), (0, _RW - 6)))
    ch = (B * _NPAD) // (_NW * _LANES)
    rows = rows.reshape(_NW, ch, _LANES, _RW)
    sidx3 = sidx.reshape(_NW, ch, _LANES)

    compact = pl.kernel(
        _sc_compact,
        out_type=jax.ShapeDtypeStruct((B * _COUT, _RW), jnp.float32),
        mesh=plsc.VectorSubcoreMesh(core_axis_name="c", subcore_axis_name="s"),
        scratch_types=[
            pltpu.VMEM((ch, _LANES), jnp.int32),
            pltpu.VMEM((ch, _LANES, _RW), jnp.float32),
            pltpu.SemaphoreType.DMA,
        ],
        compiler_params=pltpu.CompilerParams(use_tc_tiling_on_sc=False),
    )(rows, sidx3)

    cp = jnp.moveaxis(compact.reshape(B, _COUT, _RW), 1, 2)
    cp = cp[:, :6, :_CN].reshape(B, 6, _CROWS, _LANES)

    out = pl.pallas_call(
        _nms_body,
        in_specs=[pl.BlockSpec((B, 6, _CROWS, _LANES), lambda: (0, 0, 0, 0))],
        out_specs=pl.BlockSpec((B, _PROPOSALS, 4), lambda: (0, 0, 0)),
        out_shape=jax.ShapeDtypeStruct((B, _PROPOSALS, 4), jnp.float32),
    )(cp)
    return out


# speedup vs baseline: 9.1363x; 1.0984x over previous
import functools

import jax
import jax.numpy as jnp
from jax import lax
from jax.experimental import pallas as pl
from jax.experimental.pallas import tpu as pltpu
from jax.experimental.pallas import tpu_sc as plsc

_PROPOSALS = 1000
_NMS_THR = 0.7
_K = 6000
_STD = (0.1, 0.1, 0.2, 0.2)

_ROWS = 160
_LANES = 128
_NPAD = _ROWS * _LANES
_NEG = -3.0e38
_SUPPRESSED = -1.0e10

_CROWS = 48
_CN = _CROWS * _LANES
_COUT = 6400
_TRASH = 6144
_RW = 16

_NW = 32
_B = 2


def _prep_body(fg_ref, d_ref, a_ref, planes_ref, sidx_ref):
    b = pl.program_id(0)
    fg = fg_ref[0]
    ibits = lax.bitcast_convert_type(fg, jnp.int32)
    key = ibits ^ ((ibits >> 31) & jnp.int32(0x7FFFFFFF))

    ridx = lax.broadcasted_iota(jnp.int32, (_ROWS, _LANES), 0)
    cidx = lax.broadcasted_iota(jnp.int32, (_ROWS, _LANES), 1)
    gidx = ridx * _LANES + cidx

    kf = jnp.float32(_K)

    def _avg_floor(a, b2):
        return (a >> 1) + (b2 >> 1) + (a & b2 & 1)

    def _search_step(_, carry):
        lo, hi = carry
        mid = _avg_floor(lo, hi)
        cnt = jnp.sum((key >= mid).astype(jnp.float32))
        ge = cnt >= kf
        new_lo = jnp.where(ge, mid, lo)
        new_hi = jnp.where(ge, hi, mid)
        prog = hi > lo + 1
        return (jnp.where(prog, new_lo, lo), jnp.where(prog, new_hi, hi))

    lo0 = jnp.int32(-2147483647 - 1)
    hi0 = jnp.int32(2147483647)
    t, _ = lax.fori_loop(0, 33, _search_step, (lo0, hi0))

    n_gt = jnp.sum((key > t).astype(jnp.float32))
    need_ties = kf - n_gt
    is_tie = key == t

    def _jsearch(_, carry):
        jlo, jhi = carry
        jmid = (jlo + jhi) >> 1
        cnt = jnp.sum((is_tie & (gidx < jmid)).astype(jnp.float32))
        ge = cnt >= need_ties
        new_hi = jnp.where(ge, jmid, jhi)
        new_lo = jnp.where(ge, jlo, jmid + 1)
        prog = jhi > jlo
        return (jnp.where(prog, new_lo, jlo), jnp.where(prog, new_hi, jhi))

    _, jcut = lax.fori_loop(0, 16, _jsearch, (jnp.int32(0), jnp.int32(_NPAD)))

    cand = (key > t) | (is_tie & (gidx < jcut))

    d0 = d_ref[0, 0] * jnp.float32(_STD[0])
    d1 = d_ref[0, 1] * jnp.float32(_STD[1])
    d2 = d_ref[0, 2] * jnp.float32(_STD[2])
    d3 = d_ref[0, 3] * jnp.float32(_STD[3])
    a0 = a_ref[0, 0]
    a1 = a_ref[0, 1]
    a2 = a_ref[0, 2]
    a3 = a_ref[0, 3]
    height = a2 - a0
    width = a3 - a1
    cy = a0 + 0.5 * height + d0 * height
    cx = a1 + 0.5 * width + d1 * width
    height = height * jnp.exp(d2)
    width = width * jnp.exp(d3)
    y1 = jnp.clip(cy - 0.5 * height, 0.0, 1.0)
    x1 = jnp.clip(cx - 0.5 * width, 0.0, 1.0)
    y2 = jnp.clip((cy - 0.5 * height) + height, 0.0, 1.0)
    x2 = jnp.clip((cx - 0.5 * width) + width, 0.0, 1.0)
    areas = (y2 - y1) * (x2 - x1)

    candf = cand.astype(jnp.float32)
    lane_i = lax.broadcasted_iota(jnp.int32, (_LANES, _LANES), 0)
    lane_j = lax.broadcasted_iota(jnp.int32, (_LANES, _LANES), 1)
    upper_strict = (lane_i < lane_j).astype(jnp.float32)
    lane_excl = lax.dot_general(candf, upper_strict,
                                (((1,), (0,)), ((), ())),
                                preferred_element_type=jnp.float32)
    row_i = lax.broadcasted_iota(jnp.int32, (_ROWS, _ROWS), 0)
    row_j = lax.broadcasted_iota(jnp.int32, (_ROWS, _ROWS), 1)
    lower_strict = (row_i > row_j).astype(jnp.float32)
    rowtot = jnp.sum(candf, axis=1, keepdims=True)
    row_excl = lax.dot_general(lower_strict, rowtot,
                               (((1,), (0,)), ((), ())),
                               preferred_element_type=jnp.float32)
    off = (row_excl + lane_excl).astype(jnp.int32)

    trash = jnp.int32(_TRASH) + (gidx & jnp.int32(127))
    slot = jnp.where(cand, off, trash) + b * jnp.int32(_COUT)
    sidx_ref[0] = slot

    planes_ref[0, 0] = jnp.where(cand, fg, jnp.float32(_NEG))
    planes_ref[0, 1] = y1
    planes_ref[0, 2] = x1
    planes_ref[0, 3] = y2
    planes_ref[0, 4] = x2
    planes_ref[0, 5] = areas


def _sc_compact(rows_hbm, sidx_hbm, out_hbm, idx_v, rows_v, sem):
    w = lax.axis_index("s") * 2 + lax.axis_index("c")
    pltpu.sync_copy(sidx_hbm.at[w], idx_v)
    pltpu.sync_copy(rows_hbm.at[w], rows_v)
    ch = idx_v.shape[0]
    cps = [pltpu.async_copy(rows_v.at[c], out_hbm.at[idx_v.at[c]], sem)
           for c in range(ch)]
    for cp in cps:
        cp.wait()


def _key_of(f):
    ib = lax.bitcast_convert_type(f, jnp.int32)
    return ib ^ ((ib >> 31) & jnp.int32(0x7FFFFFFF))


def _rank_body(rows_ref, sp_ref, sidx2_ref):
    b = pl.program_id(0)
    lane = lax.broadcasted_iota(jnp.int32, (1, _LANES), 1)
    nblk = _CN // 512
    for bi in range(nblk):
        ks = _key_of(rows_ref[0, pl.ds(bi * 512, 512), 0:1])
        gs = (lax.broadcasted_iota(jnp.int32, (512, 1), 0)
              + jnp.int32(bi * 512))

        def _acc(j, acc):
            kl = _key_of(sp_ref[0, 0, pl.ds(j, 1), :])
            gj = lane + j * _LANES
            cmp = (kl > ks) | ((kl == ks) & (gj < gs))
            cmp = cmp & (gj < _K)
            return acc + cmp.astype(jnp.float32)

        acc = lax.fori_loop(0, _CROWS, _acc,
                            jnp.zeros((512, _LANES), jnp.float32))
        rank = jnp.sum(acc, axis=1, keepdims=True).astype(jnp.int32)
        trash = jnp.int32(_TRASH) + (gs & jnp.int32(127))
        slot = jnp.where(gs < _K, rank, trash) + b * jnp.int32(_COUT)
        sidx2_ref[0, pl.ds(bi * 512, 512), 0:1] = slot


def _nms_body(p_ref, rows_ref, out_ref):
    gidx = (lax.broadcasted_iota(jnp.int32, (_CROWS, _LANES), 0) * _LANES
            + lax.broadcasted_iota(jnp.int32, (_CROWS, _LANES), 1))

    st = [(p_ref[b, 1], p_ref[b, 2], p_ref[b, 3], p_ref[b, 4], p_ref[b, 5])
          for b in range(_B)]
    io16 = lax.broadcasted_iota(jnp.int32, (1, _RW), 1)

    def _step(p, alives):
        new_alives = []
        for b in range(_B):
            alive = alives[b]
            y1, x1, y2, x2, areas = st[b]
            idxsel = jnp.min(jnp.where(alive > 0.5, gidx, jnp.int32(_CN)))
            valid = idxsel < _K
            row16 = rows_ref[b, pl.ds(idxsel, 1), :]
            by1 = jnp.sum(jnp.where(io16 == 1, row16, 0.0))
            bx1 = jnp.sum(jnp.where(io16 == 2, row16, 0.0))
            by2 = jnp.sum(jnp.where(io16 == 3, row16, 0.0))
            bx2 = jnp.sum(jnp.where(io16 == 4, row16, 0.0))
            barea = (by2 - by1) * (bx2 - bx1)
            yy1 = jnp.maximum(by1, y1)
            xx1 = jnp.maximum(bx1, x1)
            yy2 = jnp.minimum(by2, y2)
            xx2 = jnp.minimum(bx2, x2)
            inter = jnp.maximum(yy2 - yy1, 0.0) * jnp.maximum(xx2 - xx1, 0.0)
            iou = inter / (barea + areas - inter + 1e-8)
            suppress = (iou > _NMS_THR) | (gidx == idxsel)
            alive = alive * jnp.where(valid & suppress, 0.0, 1.0)
            out_ref[b, pl.ds(p, 1), :] = jnp.where(valid, row16[:, 1:5], 0.0)
            new_alives.append(alive)
        return tuple(new_alives)

    alive0 = (gidx < _K).astype(jnp.float32)
    lax.fori_loop(0, _PROPOSALS, _step, (alive0,) * _B)


def kernel(scores, deltas, anchors):
    B, N, _ = scores.shape
    fg = scores[:, :, 1]
    pad = _NPAD - N
    fg = jnp.pad(fg, ((0, 0), (0, pad)), constant_values=_NEG)
    fg = fg.reshape(B, _ROWS, _LANES)
    d = jnp.moveaxis(deltas, 2, 1)
    a = jnp.moveaxis(anchors, 2, 1)
    d = jnp.pad(d, ((0, 0), (0, 0), (0, pad)))
    a = jnp.pad(a, ((0, 0), (0, 0), (0, pad)))
    d = d.reshape(B, 4, _ROWS, _LANES)
    a = a.reshape(B, 4, _ROWS, _LANES)

    planes, sidx = pl.pallas_call(
        _prep_body,
        grid=(B,),
        in_specs=[
            pl.BlockSpec((1, _ROWS, _LANES), lambda b: (b, 0, 0)),
            pl.BlockSpec((1, 4, _ROWS, _LANES), lambda b: (b, 0, 0, 0)),
            pl.BlockSpec((1, 4, _ROWS, _LANES), lambda b: (b, 0, 0, 0)),
        ],
        out_specs=[
            pl.BlockSpec((1, 6, _ROWS, _LANES), lambda b: (b, 0, 0, 0)),
            pl.BlockSpec((1, _ROWS, _LANES), lambda b: (b, 0, 0)),
        ],
        out_shape=[
            jax.ShapeDtypeStruct((B, 6, _ROWS, _LANES), jnp.float32),
            jax.ShapeDtypeStruct((B, _ROWS, _LANES), jnp.int32),
        ],
    )(fg, d, a)

    def _sc_scatter(rows_flat, sidx_flat):
        nrow = rows_flat.shape[0]
        ch = nrow // (_NW * _LANES)
        rows3 = rows_flat.reshape(_NW, ch, _LANES, _RW)
        sidx3 = sidx_flat.reshape(_NW, ch, _LANES)
        return pl.kernel(
            _sc_compact,
            out_type=jax.ShapeDtypeStruct((B * _COUT, _RW), jnp.float32),
            mesh=plsc.VectorSubcoreMesh(core_axis_name="c",
                                        subcore_axis_name="s"),
            scratch_types=[
                pltpu.VMEM((ch, _LANES), jnp.int32),
                pltpu.VMEM((ch, _LANES, _RW), jnp.float32),
                pltpu.SemaphoreType.DMA,
            ],
            compiler_params=pltpu.CompilerParams(use_tc_tiling_on_sc=False),
        )(rows3, sidx3)

    rows = jnp.moveaxis(planes.reshape(B, 6, _NPAD), 1, 2)
    rows = jnp.pad(rows, ((0, 0), (0, 0), (0, _RW - 6)))
    compact = _sc_scatter(rows.reshape(B * _NPAD, _RW), sidx.reshape(-1))

    rows1 = compact.reshape(B, _COUT, _RW)
    sp = jnp.moveaxis(rows1[:, :_CN, 0:1], 1, 2)
    sp = sp.reshape(B, 1, _CROWS, _LANES)
    sidx2 = pl.pallas_call(
        _rank_body,
        grid=(B,),
        in_specs=[
            pl.BlockSpec((1, _COUT, _RW), lambda b: (b, 0, 0)),
            pl.BlockSpec((1, 1, _CROWS, _LANES), lambda b: (b, 0, 0, 0)),
        ],
        out_specs=pl.BlockSpec((1, _CN, 1), lambda b: (b, 0, 0)),
        out_shape=jax.ShapeDtypeStruct((B, _CN, 1), jnp.int32),
    )(rows1, sp)

    compact2 = _sc_scatter(rows1[:, :_CN].reshape(B * _CN, _RW),
                           sidx2.reshape(-1))

    rows2 = compact2.reshape(B, _COUT, _RW)
    cp2 = jnp.moveaxis(rows2, 1, 2)[:, 1:6, :_CN]
    cp2 = cp2.reshape(B, 5, _CROWS, _LANES)
    cp2 = jnp.pad(cp2, ((0, 0), (1, 0), (0, 0), (0, 0)))

    out = pl.pallas_call(
        _nms_body,
        in_specs=[
            pl.BlockSpec((B, 6, _CROWS, _LANES), lambda: (0, 0, 0, 0)),
            pl.BlockSpec((B, _COUT, _RW), lambda: (0, 0, 0)),
        ],
        out_specs=pl.BlockSpec((B, _PROPOSALS, 4), lambda: (0, 0, 0)),
        out_shape=jax.ShapeDtypeStruct((B, _PROPOSALS, 4), jnp.float32),
    )(cp2, rows2)
    return out


# speedup vs baseline: 10.0259x; 1.0974x over previous
import functools

import jax
import jax.numpy as jnp
from jax import lax
from jax.experimental import pallas as pl
from jax.experimental.pallas import tpu as pltpu
from jax.experimental.pallas import tpu_sc as plsc

_PROPOSALS = 1000
_NMS_THR = 0.7
_K = 6000
_STD = (0.1, 0.1, 0.2, 0.2)

_ROWS = 160
_LANES = 128
_NPAD = _ROWS * _LANES
_NEG = -3.0e38
_SUPPRESSED = -1.0e10

_CROWS = 48
_CN = _CROWS * _LANES
_COUT = 6400
_TRASH = 6144
_RW = 16

_NW = 32
_B = 2
_ZFILL = 1024
_AOUT = 1152


def _prep_body(fg_ref, d_ref, a_ref, planes_ref, sidx_ref):
    b = pl.program_id(0)
    fg = fg_ref[0]
    ibits = lax.bitcast_convert_type(fg, jnp.int32)
    key = ibits ^ ((ibits >> 31) & jnp.int32(0x7FFFFFFF))

    ridx = lax.broadcasted_iota(jnp.int32, (_ROWS, _LANES), 0)
    cidx = lax.broadcasted_iota(jnp.int32, (_ROWS, _LANES), 1)
    gidx = ridx * _LANES + cidx

    kf = jnp.float32(_K)

    def _avg_floor(a, b2):
        return (a >> 1) + (b2 >> 1) + (a & b2 & 1)

    def _search_step(_, carry):
        lo, hi = carry
        mid = _avg_floor(lo, hi)
        cnt = jnp.sum((key >= mid).astype(jnp.float32))
        ge = cnt >= kf
        new_lo = jnp.where(ge, mid, lo)
        new_hi = jnp.where(ge, hi, mid)
        prog = hi > lo + 1
        return (jnp.where(prog, new_lo, lo), jnp.where(prog, new_hi, hi))

    lo0 = jnp.int32(-2147483647 - 1)
    hi0 = jnp.int32(2147483647)
    t, _ = lax.fori_loop(0, 33, _search_step, (lo0, hi0))

    n_gt = jnp.sum((key > t).astype(jnp.float32))
    need_ties = kf - n_gt
    is_tie = key == t

    def _jsearch(_, carry):
        jlo, jhi = carry
        jmid = (jlo + jhi) >> 1
        cnt = jnp.sum((is_tie & (gidx < jmid)).astype(jnp.float32))
        ge = cnt >= need_ties
        new_hi = jnp.where(ge, jmid, jhi)
        new_lo = jnp.where(ge, jlo, jmid + 1)
        prog = jhi > jlo
        return (jnp.where(prog, new_lo, jlo), jnp.where(prog, new_hi, jhi))

    _, jcut = lax.fori_loop(0, 16, _jsearch, (jnp.int32(0), jnp.int32(_NPAD)))

    cand = (key > t) | (is_tie & (gidx < jcut))

    d0 = d_ref[0, 0] * jnp.float32(_STD[0])
    d1 = d_ref[0, 1] * jnp.float32(_STD[1])
    d2 = d_ref[0, 2] * jnp.float32(_STD[2])
    d3 = d_ref[0, 3] * jnp.float32(_STD[3])
    a0 = a_ref[0, 0]
    a1 = a_ref[0, 1]
    a2 = a_ref[0, 2]
    a3 = a_ref[0, 3]
    height = a2 - a0
    width = a3 - a1
    cy = a0 + 0.5 * height + d0 * height
    cx = a1 + 0.5 * width + d1 * width
    height = height * jnp.exp(d2)
    width = width * jnp.exp(d3)
    y1 = jnp.clip(cy - 0.5 * height, 0.0, 1.0)
    x1 = jnp.clip(cx - 0.5 * width, 0.0, 1.0)
    y2 = jnp.clip((cy - 0.5 * height) + height, 0.0, 1.0)
    x2 = jnp.clip((cx - 0.5 * width) + width, 0.0, 1.0)
    areas = (y2 - y1) * (x2 - x1)

    candf = cand.astype(jnp.float32)
    lane_i = lax.broadcasted_iota(jnp.int32, (_LANES, _LANES), 0)
    lane_j = lax.broadcasted_iota(jnp.int32, (_LANES, _LANES), 1)
    upper_strict = (lane_i < lane_j).astype(jnp.float32)
    lane_excl = lax.dot_general(candf, upper_strict,
                                (((1,), (0,)), ((), ())),
                                preferred_element_type=jnp.float32)
    row_i = lax.broadcasted_iota(jnp.int32, (_ROWS, _ROWS), 0)
    row_j = lax.broadcasted_iota(jnp.int32, (_ROWS, _ROWS), 1)
    lower_strict = (row_i > row_j).astype(jnp.float32)
    rowtot = jnp.sum(candf, axis=1, keepdims=True)
    row_excl = lax.dot_general(lower_strict, rowtot,
                               (((1,), (0,)), ((), ())),
                               preferred_element_type=jnp.float32)
    off = (row_excl + lane_excl).astype(jnp.int32)

    trash = jnp.int32(_TRASH) + (gidx & jnp.int32(127))
    slot = jnp.where(cand, off, trash) + b * jnp.int32(_COUT)
    sidx_ref[0] = slot

    planes_ref[0, 0] = jnp.where(cand, fg, jnp.float32(_NEG))
    planes_ref[0, 1] = y1
    planes_ref[0, 2] = x1
    planes_ref[0, 3] = y2
    planes_ref[0, 4] = x2
    planes_ref[0, 5] = areas


def _sc_compact(rows_hbm, sidx_hbm, out_hbm, idx_v, rows_v, sem):
    w = lax.axis_index("s") * 2 + lax.axis_index("c")
    pltpu.sync_copy(sidx_hbm.at[w], idx_v)
    pltpu.sync_copy(rows_hbm.at[w], rows_v)
    ch = idx_v.shape[0]
    cps = [pltpu.async_copy(rows_v.at[c], out_hbm.at[idx_v.at[c]], sem)
           for c in range(ch)]
    for cp in cps:
        cp.wait()


def _key_of(f):
    ib = lax.bitcast_convert_type(f, jnp.int32)
    return ib ^ ((ib >> 31) & jnp.int32(0x7FFFFFFF))


def _rank_body(rows_ref, sp_ref, sidx2_ref):
    b = pl.program_id(0)
    lane = lax.broadcasted_iota(jnp.int32, (1, _LANES), 1)
    nblk = _CN // 512
    for bi in range(nblk):
        ks = _key_of(rows_ref[0, pl.ds(bi * 512, 512), 0:1])
        gs = (lax.broadcasted_iota(jnp.int32, (512, 1), 0)
              + jnp.int32(bi * 512))

        def _acc(j, acc):
            kl = _key_of(sp_ref[0, 0, pl.ds(j, 1), :])
            gj = lane + j * _LANES
            cmp = (kl > ks) | ((kl == ks) & (gj < gs))
            cmp = cmp & (gj < _K)
            return acc + cmp.astype(jnp.float32)

        acc = lax.fori_loop(0, _CROWS, _acc,
                            jnp.zeros((512, _LANES), jnp.float32))
        rank = jnp.sum(acc, axis=1, keepdims=True).astype(jnp.int32)
        trash = jnp.int32(_TRASH) + (gs & jnp.int32(127))
        slot = jnp.where(gs < _K, rank, trash) + b * jnp.int32(_COUT)
        sidx2_ref[0, pl.ds(bi * 512, 512), 0:1] = slot


def _banded_body(p_ref, rows_ref, sidx3_ref, kr_ref):
    b = pl.program_id(0)
    lane = lax.broadcasted_iota(jnp.int32, (1, _LANES), 1)
    eye = (lax.broadcasted_iota(jnp.int32, (_LANES, _LANES), 0)
           == lax.broadcasted_iota(jnp.int32, (_LANES, _LANES), 1)
           ).astype(jnp.float32)
    kr_ref[...] = jnp.zeros((_CN, 1), jnp.float32)

    def _iou_block(qrows, nq, pc):
        y1q = qrows[:, 1:2]
        x1q = qrows[:, 2:3]
        y2q = qrows[:, 3:4]
        x2q = qrows[:, 4:5]
        aq = qrows[:, 5:6]
        y1p, x1p, y2p, x2p, ap = pc
        yy1 = jnp.maximum(y1q, jnp.broadcast_to(y1p, (nq, _LANES)))
        xx1 = jnp.maximum(x1q, jnp.broadcast_to(x1p, (nq, _LANES)))
        yy2 = jnp.minimum(y2q, jnp.broadcast_to(y2p, (nq, _LANES)))
        xx2 = jnp.minimum(x2q, jnp.broadcast_to(x2p, (nq, _LANES)))
        inter = jnp.maximum(yy2 - yy1, 0.0) * jnp.maximum(xx2 - xx1, 0.0)
        return inter / (aq + ap - inter + 1e-8)

    kept_bands = []
    for i in range(_CROWS):
        pc = tuple(p_ref[0, k, pl.ds(i, 1), :] for k in range(1, 6))
        gi = lane + i * _LANES

        nqb = (i + 3) // 4
        qiota = lax.broadcasted_iota(jnp.int32, (512, 1), 0)

        def _qstep(qb, acc, pc=pc, i=i, qiota=qiota):
            qrows = rows_ref[0, pl.ds(qb * 512, 512), :]
            keptq = kr_ref[pl.ds(qb * 512, 512), :]
            iou = _iou_block(qrows, 512, pc)
            m = ((iou > _NMS_THR) & (keptq > 0.5)
                 & ((qiota + qb * 512) < i * _LANES))
            return jnp.maximum(acc, jnp.max(m.astype(jnp.float32), axis=0,
                                            keepdims=True))

        cross = jnp.zeros((1, _LANES), jnp.float32)
        if i > 0:
            cross = lax.fori_loop(0, nqb, _qstep, cross)

        init = jnp.where((cross < 0.5) & (gi < _K), 1.0, 0.0)

        qrows_i = rows_ref[0, pl.ds(i * _LANES, _LANES), :]
        iou_ii = _iou_block(qrows_i, _LANES, pc)
        liota = lax.broadcasted_iota(jnp.int32, (_LANES, 1), 0)
        Sf = ((iou_ii > _NMS_THR) & (liota < lane)).astype(jnp.float32)

        def _fix_cond(state):
            return state[1]

        def _fix_body(state, Sf=Sf, init=init):
            kept, _ = state
            kept_r = lax.dot_general(eye, kept, (((1,), (1,)), ((), ())),
                                     preferred_element_type=jnp.float32)
            supp = jnp.max(Sf * kept_r, axis=0, keepdims=True)
            new = jnp.where(supp > 0.5, 0.0, init)
            changed = jnp.sum(jnp.abs(new - kept)) > 0.0
            return (new, changed)

        kept_i, _ = lax.while_loop(_fix_cond, _fix_body,
                                   (init, jnp.bool_(True)))
        kept_bands.append(kept_i)
        kr_ref[pl.ds(i * _LANES, _LANES), :] = lax.dot_general(
            eye, kept_i, (((1,), (1,)), ((), ())),
            preferred_element_type=jnp.float32)

    kept = jnp.concatenate(kept_bands, axis=0)

    lane_i = lax.broadcasted_iota(jnp.int32, (_LANES, _LANES), 0)
    lane_j = lax.broadcasted_iota(jnp.int32, (_LANES, _LANES), 1)
    upper_strict = (lane_i < lane_j).astype(jnp.float32)
    lane_excl = lax.dot_general(kept, upper_strict, (((1,), (0,)), ((), ())),
                                preferred_element_type=jnp.float32)
    row_i = lax.broadcasted_iota(jnp.int32, (_CROWS, _CROWS), 0)
    row_j = lax.broadcasted_iota(jnp.int32, (_CROWS, _CROWS), 1)
    lower_strict = (row_i > row_j).astype(jnp.float32)
    rowtot = jnp.sum(kept, axis=1, keepdims=True)
    row_excl = lax.dot_general(lower_strict, rowtot, (((1,), (0,)), ((), ())),
                               preferred_element_type=jnp.float32)
    rank = (row_excl + lane_excl).astype(jnp.int32)
    gidx = (lax.broadcasted_iota(jnp.int32, (_CROWS, _LANES), 0) * _LANES
            + lax.broadcasted_iota(jnp.int32, (_CROWS, _LANES), 1))
    trash = jnp.int32(_ZFILL) + (gidx & jnp.int32(127))
    slot = jnp.where((kept > 0.5) & (rank < _ZFILL), rank, trash)
    sidx3_ref[0] = slot + b * jnp.int32(_AOUT)


def _sc_assemble(rows_hbm, sidx_hbm, out_hbm, idx_v, rows_v, zero_v, sem):
    c = lax.axis_index("c")
    s = lax.axis_index("s")
    for r in range(64):
        zero_v[r, :] = jnp.zeros((_RW,), jnp.float32)
    base = c * _AOUT + s * 64
    pltpu.sync_copy(zero_v, out_hbm.at[pl.ds(base, 64)])
    plsc.subcore_barrier()
    w = s * 2 + c
    pltpu.sync_copy(sidx_hbm.at[w], idx_v)
    pltpu.sync_copy(rows_hbm.at[w], rows_v)
    ch = idx_v.shape[0]
    cps = [pltpu.async_copy(rows_v.at[cc], out_hbm.at[idx_v.at[cc]], sem)
           for cc in range(ch)]
    for cp in cps:
        cp.wait()


def kernel(scores, deltas, anchors):
    B, N, _ = scores.shape
    fg = scores[:, :, 1]
    pad = _NPAD - N
    fg = jnp.pad(fg, ((0, 0), (0, pad)), constant_values=_NEG)
    fg = fg.reshape(B, _ROWS, _LANES)
    d = jnp.moveaxis(deltas, 2, 1)
    a = jnp.moveaxis(anchors, 2, 1)
    d = jnp.pad(d, ((0, 0), (0, 0), (0, pad)))
    a = jnp.pad(a, ((0, 0), (0, 0), (0, pad)))
    d = d.reshape(B, 4, _ROWS, _LANES)
    a = a.reshape(B, 4, _ROWS, _LANES)

    planes, sidx = pl.pallas_call(
        _prep_body,
        grid=(B,),
        in_specs=[
            pl.BlockSpec((1, _ROWS, _LANES), lambda b: (b, 0, 0)),
            pl.BlockSpec((1, 4, _ROWS, _LANES), lambda b: (b, 0, 0, 0)),
            pl.BlockSpec((1, 4, _ROWS, _LANES), lambda b: (b, 0, 0, 0)),
        ],
        out_specs=[
            pl.BlockSpec((1, 6, _ROWS, _LANES), lambda b: (b, 0, 0, 0)),
            pl.BlockSpec((1, _ROWS, _LANES), lambda b: (b, 0, 0)),
        ],
        out_shape=[
            jax.ShapeDtypeStruct((B, 6, _ROWS, _LANES), jnp.float32),
            jax.ShapeDtypeStruct((B, _ROWS, _LANES), jnp.int32),
        ],
    )(fg, d, a)

    def _sc_scatter(rows_flat, sidx_flat):
        nrow = rows_flat.shape[0]
        ch = nrow // (_NW * _LANES)
        rows3 = rows_flat.reshape(_NW, ch, _LANES, _RW)
        sidx3 = sidx_flat.reshape(_NW, ch, _LANES)
        return pl.kernel(
            _sc_compact,
            out_type=jax.ShapeDtypeStruct((B * _COUT, _RW), jnp.float32),
            mesh=plsc.VectorSubcoreMesh(core_axis_name="c",
                                        subcore_axis_name="s"),
            scratch_types=[
                pltpu.VMEM((ch, _LANES), jnp.int32),
                pltpu.VMEM((ch, _LANES, _RW), jnp.float32),
                pltpu.SemaphoreType.DMA,
            ],
            compiler_params=pltpu.CompilerParams(use_tc_tiling_on_sc=False),
        )(rows3, sidx3)

    rows = jnp.moveaxis(planes.reshape(B, 6, _NPAD), 1, 2)
    rows = jnp.pad(rows, ((0, 0), (0, 0), (0, _RW - 6)))
    compact = _sc_scatter(rows.reshape(B * _NPAD, _RW), sidx.reshape(-1))

    rows1 = compact.reshape(B, _COUT, _RW)
    sp = jnp.moveaxis(rows1[:, :_CN, 0:1], 1, 2)
    sp = sp.reshape(B, 1, _CROWS, _LANES)
    sidx2 = pl.pallas_call(
        _rank_body,
        grid=(B,),
        in_specs=[
            pl.BlockSpec((1, _COUT, _RW), lambda b: (b, 0, 0)),
            pl.BlockSpec((1, 1, _CROWS, _LANES), lambda b: (b, 0, 0, 0)),
        ],
        out_specs=pl.BlockSpec((1, _CN, 1), lambda b: (b, 0, 0)),
        out_shape=jax.ShapeDtypeStruct((B, _CN, 1), jnp.int32),
    )(rows1, sp)

    compact2 = _sc_scatter(rows1[:, :_CN].reshape(B * _CN, _RW),
                           sidx2.reshape(-1))

    rows2 = compact2.reshape(B, _COUT, _RW)
    cp2 = jnp.moveaxis(rows2, 1, 2)[:, 1:6, :_CN]
    cp2 = cp2.reshape(B, 5, _CROWS, _LANES)
    cp2 = jnp.pad(cp2, ((0, 0), (1, 0), (0, 0), (0, 0)))

    sidx3 = pl.pallas_call(
        _banded_body,
        grid=(B,),
        in_specs=[
            pl.BlockSpec((1, 6, _CROWS, _LANES), lambda b: (b, 0, 0, 0)),
            pl.BlockSpec((1, _COUT, _RW), lambda b: (b, 0, 0)),
        ],
        out_specs=pl.BlockSpec((1, _CROWS, _LANES), lambda b: (b, 0, 0)),
        out_shape=jax.ShapeDtypeStruct((B, _CROWS, _LANES), jnp.int32),
        scratch_shapes=[pltpu.VMEM((_CN, 1), jnp.float32)],
    )(cp2, rows2)

    arows = rows2[:, :_CN].reshape(B, 16, 3, _LANES, _RW)
    arows = jnp.moveaxis(arows, 0, 1).reshape(_NW, 3, _LANES, _RW)
    aidx = sidx3.reshape(B, 16, 3, _LANES)
    aidx = jnp.moveaxis(aidx, 0, 1).reshape(_NW, 3, _LANES)
    out3 = pl.kernel(
        _sc_assemble,
        out_type=jax.ShapeDtypeStruct((B * _AOUT, _RW), jnp.float32),
        mesh=plsc.VectorSubcoreMesh(core_axis_name="c", subcore_axis_name="s"),
        scratch_types=[
            pltpu.VMEM((3, _LANES), jnp.int32),
            pltpu.VMEM((3, _LANES, _RW), jnp.float32),
            pltpu.VMEM((64, _RW), jnp.float32),
            pltpu.SemaphoreType.DMA,
        ],
        compiler_params=pltpu.CompilerParams(use_tc_tiling_on_sc=False),
    )(arows, aidx)
    return out3.reshape(B, _AOUT, _RW)[:, :_PROPOSALS, 1:5]


# speedup vs baseline: 12.0151x; 1.1984x over previous
import functools

import jax
import jax.numpy as jnp
from jax import lax
from jax.experimental import pallas as pl
from jax.experimental.pallas import tpu as pltpu
from jax.experimental.pallas import tpu_sc as plsc

_PROPOSALS = 1000
_NMS_THR = 0.7
_K = 6000
_STD = (0.1, 0.1, 0.2, 0.2)

_ROWS = 160
_LANES = 128
_NPAD = _ROWS * _LANES
_NEG = -3.0e38
_SUPPRESSED = -1.0e10

_CROWS = 48
_CN = _CROWS * _LANES
_COUT = 6400
_TRASH = 6144
_RW = 16

_NW = 32
_B = 2
_ZFILL = 1024
_AOUT = 1152


def _prep_body(fg_ref, d_ref, a_ref, planes_ref, sidx_ref):
    b = pl.program_id(0)
    fg = fg_ref[0]
    ibits = lax.bitcast_convert_type(fg, jnp.int32)
    key = ibits ^ ((ibits >> 31) & jnp.int32(0x7FFFFFFF))

    ridx = lax.broadcasted_iota(jnp.int32, (_ROWS, _LANES), 0)
    cidx = lax.broadcasted_iota(jnp.int32, (_ROWS, _LANES), 1)
    gidx = ridx * _LANES + cidx

    kf = jnp.float32(_K)

    def _avg_floor(a, b2):
        return (a >> 1) + (b2 >> 1) + (a & b2 & 1)

    def _search_step(_, carry):
        lo, hi = carry
        mid = _avg_floor(lo, hi)
        cnt = jnp.sum((key >= mid).astype(jnp.float32))
        ge = cnt >= kf
        new_lo = jnp.where(ge, mid, lo)
        new_hi = jnp.where(ge, hi, mid)
        prog = hi > lo + 1
        return (jnp.where(prog, new_lo, lo), jnp.where(prog, new_hi, hi))

    lo0 = jnp.int32(-2147483647 - 1)
    hi0 = jnp.int32(2147483647)
    t, _ = lax.fori_loop(0, 33, _search_step, (lo0, hi0))

    n_gt = jnp.sum((key > t).astype(jnp.float32))
    need_ties = kf - n_gt
    is_tie = key == t

    def _jsearch(_, carry):
        jlo, jhi = carry
        jmid = (jlo + jhi) >> 1
        cnt = jnp.sum((is_tie & (gidx < jmid)).astype(jnp.float32))
        ge = cnt >= need_ties
        new_hi = jnp.where(ge, jmid, jhi)
        new_lo = jnp.where(ge, jlo, jmid + 1)
        prog = jhi > jlo
        return (jnp.where(prog, new_lo, jlo), jnp.where(prog, new_hi, jhi))

    _, jcut = lax.fori_loop(0, 16, _jsearch, (jnp.int32(0), jnp.int32(_NPAD)))

    cand = (key > t) | (is_tie & (gidx < jcut))

    d0 = d_ref[0, 0] * jnp.float32(_STD[0])
    d1 = d_ref[0, 1] * jnp.float32(_STD[1])
    d2 = d_ref[0, 2] * jnp.float32(_STD[2])
    d3 = d_ref[0, 3] * jnp.float32(_STD[3])
    a0 = a_ref[0, 0]
    a1 = a_ref[0, 1]
    a2 = a_ref[0, 2]
    a3 = a_ref[0, 3]
    height = a2 - a0
    width = a3 - a1
    cy = a0 + 0.5 * height + d0 * height
    cx = a1 + 0.5 * width + d1 * width
    height = height * jnp.exp(d2)
    width = width * jnp.exp(d3)
    y1 = jnp.clip(cy - 0.5 * height, 0.0, 1.0)
    x1 = jnp.clip(cx - 0.5 * width, 0.0, 1.0)
    y2 = jnp.clip((cy - 0.5 * height) + height, 0.0, 1.0)
    x2 = jnp.clip((cx - 0.5 * width) + width, 0.0, 1.0)
    areas = (y2 - y1) * (x2 - x1)

    candf = cand.astype(jnp.float32)
    lane_i = lax.broadcasted_iota(jnp.int32, (_LANES, _LANES), 0)
    lane_j = lax.broadcasted_iota(jnp.int32, (_LANES, _LANES), 1)
    upper_strict = (lane_i < lane_j).astype(jnp.float32)
    lane_excl = lax.dot_general(candf, upper_strict,
                                (((1,), (0,)), ((), ())),
                                preferred_element_type=jnp.float32)
    row_i = lax.broadcasted_iota(jnp.int32, (_ROWS, _ROWS), 0)
    row_j = lax.broadcasted_iota(jnp.int32, (_ROWS, _ROWS), 1)
    lower_strict = (row_i > row_j).astype(jnp.float32)
    rowtot = jnp.sum(candf, axis=1, keepdims=True)
    row_excl = lax.dot_general(lower_strict, rowtot,
                               (((1,), (0,)), ((), ())),
                               preferred_element_type=jnp.float32)
    off = (row_excl + lane_excl).astype(jnp.int32)

    trash = jnp.int32(_TRASH) + (gidx & jnp.int32(127))
    slot = jnp.where(cand, off, trash) + b * jnp.int32(_COUT)
    sidx_ref[0] = slot

    planes_ref[0, 0] = jnp.where(cand, fg, jnp.float32(_NEG))
    planes_ref[0, 1] = y1
    planes_ref[0, 2] = x1
    planes_ref[0, 3] = y2
    planes_ref[0, 4] = x2
    planes_ref[0, 5] = areas


def _sc_compact(rows_hbm, sidx_hbm, out_hbm, idx_v, rows_v, sem):
    w = lax.axis_index("s") * 2 + lax.axis_index("c")
    pltpu.sync_copy(sidx_hbm.at[w], idx_v)
    pltpu.sync_copy(rows_hbm.at[w], rows_v)
    ch = idx_v.shape[0]
    cps = [pltpu.async_copy(rows_v.at[c], out_hbm.at[idx_v.at[c]], sem)
           for c in range(ch)]
    for cp in cps:
        cp.wait()


def _key_of(f):
    ib = lax.bitcast_convert_type(f, jnp.int32)
    return ib ^ ((ib >> 31) & jnp.int32(0x7FFFFFFF))


def _rank_body(rows_ref, sp_ref, sidx2_ref):
    b = pl.program_id(0)
    lane = lax.broadcasted_iota(jnp.int32, (1, _LANES), 1)
    nblk = _CN // 512
    for bi in range(nblk):
        ks = _key_of(rows_ref[0, pl.ds(bi * 512, 512), 0:1])
        gs = (lax.broadcasted_iota(jnp.int32, (512, 1), 0)
              + jnp.int32(bi * 512))

        def _acc(j4, acc):
            for u in range(4):
                j = j4 * 4 + u
                kl = _key_of(sp_ref[0, 0, pl.ds(j, 1), :])
                gj = lane + j * _LANES
                cmp = (kl > ks) | ((kl == ks) & (gj < gs))
                cmp = cmp & (gj < _K)
                acc = acc + cmp.astype(jnp.float32)
            return acc

        acc = lax.fori_loop(0, _CROWS // 4, _acc,
                            jnp.zeros((512, _LANES), jnp.float32))
        rank = jnp.sum(acc, axis=1, keepdims=True).astype(jnp.int32)
        trash = jnp.int32(_TRASH) + (gs & jnp.int32(127))
        slot = jnp.where(gs < _K, rank, trash) + b * jnp.int32(_COUT)
        sidx2_ref[0, pl.ds(bi * 512, 512), 0:1] = slot


def _banded_body(p_ref, rows_ref, sidx3_ref, kr_ref):
    lane = lax.broadcasted_iota(jnp.int32, (1, _LANES), 1)
    eye = (lax.broadcasted_iota(jnp.int32, (_LANES, _LANES), 0)
           == lax.broadcasted_iota(jnp.int32, (_LANES, _LANES), 1)
           ).astype(jnp.float32)
    kr_ref[...] = jnp.zeros((_B * _CN, 1), jnp.float32)

    def _iou_block(qrows, nq, pc):
        y1q = qrows[:, 1:2]
        x1q = qrows[:, 2:3]
        y2q = qrows[:, 3:4]
        x2q = qrows[:, 4:5]
        aq = qrows[:, 5:6]
        y1p, x1p, y2p, x2p, ap = pc
        yy1 = jnp.maximum(y1q, jnp.broadcast_to(y1p, (nq, _LANES)))
        xx1 = jnp.maximum(x1q, jnp.broadcast_to(x1p, (nq, _LANES)))
        yy2 = jnp.minimum(y2q, jnp.broadcast_to(y2p, (nq, _LANES)))
        xx2 = jnp.minimum(x2q, jnp.broadcast_to(x2p, (nq, _LANES)))
        inter = jnp.maximum(yy2 - yy1, 0.0) * jnp.maximum(xx2 - xx1, 0.0)
        return inter / (aq + ap - inter + 1e-8)

    kept_bands = [[] for _ in range(_B)]
    qiota = lax.broadcasted_iota(jnp.int32, (512, 1), 0)
    liota = lax.broadcasted_iota(jnp.int32, (_LANES, 1), 0)
    for i in range(_CROWS):
        pcs = [tuple(p_ref[b, k, pl.ds(i, 1), :] for k in range(1, 6))
               for b in range(_B)]
        gi = lane + i * _LANES

        nqb = (i + 3) // 4

        def _qstep(qb, accs, pcs=pcs, i=i):
            outs = []
            for b in range(_B):
                qrows = rows_ref[b, pl.ds(qb * 512, 512), :]
                keptq = kr_ref[pl.ds(b * _CN + qb * 512, 512), :]
                iou = _iou_block(qrows, 512, pcs[b])
                m = ((iou > _NMS_THR) & (keptq > 0.5)
                     & ((qiota + qb * 512) < i * _LANES))
                outs.append(jnp.maximum(
                    accs[b], jnp.max(m.astype(jnp.float32), axis=0,
                                     keepdims=True)))
            return tuple(outs)

        cross = (jnp.zeros((1, _LANES), jnp.float32),) * _B
        if i > 0:
            cross = lax.fori_loop(0, nqb, _qstep, cross)

        inits, Sfs = [], []
        for b in range(_B):
            inits.append(jnp.where((cross[b] < 0.5) & (gi < _K), 1.0, 0.0))
            qrows_i = rows_ref[b, pl.ds(i * _LANES, _LANES), :]
            iou_ii = _iou_block(qrows_i, _LANES, pcs[b])
            Sfs.append(((iou_ii > _NMS_THR)
                        & (liota < lane)).astype(jnp.float32))

        def _fix_cond(state):
            return state[-1]

        def _fix_body(state, Sfs=Sfs, inits=inits):
            kept = state[:_B]
            news, changed = [], jnp.bool_(False)
            for b in range(_B):
                kept_r = lax.dot_general(eye, kept[b],
                                         (((1,), (1,)), ((), ())),
                                         preferred_element_type=jnp.float32)
                supp = jnp.max(Sfs[b] * kept_r, axis=0, keepdims=True)
                new = jnp.where(supp > 0.5, 0.0, inits[b])
                changed = changed | (jnp.sum(jnp.abs(new - kept[b])) > 0.0)
                news.append(new)
            return (*news, changed)

        fixed = lax.while_loop(_fix_cond, _fix_body,
                               (*inits, jnp.bool_(True)))
        for b in range(_B):
            kept_bands[b].append(fixed[b])
            kr_ref[pl.ds(b * _CN + i * _LANES, _LANES), :] = lax.dot_general(
                eye, fixed[b], (((1,), (1,)), ((), ())),
                preferred_element_type=jnp.float32)

    lane_i = lax.broadcasted_iota(jnp.int32, (_LANES, _LANES), 0)
    lane_j = lax.broadcasted_iota(jnp.int32, (_LANES, _LANES), 1)
    upper_strict = (lane_i < lane_j).astype(jnp.float32)
    row_i = lax.broadcasted_iota(jnp.int32, (_CROWS, _CROWS), 0)
    row_j = lax.broadcasted_iota(jnp.int32, (_CROWS, _CROWS), 1)
    lower_strict = (row_i > row_j).astype(jnp.float32)
    gidx = (lax.broadcasted_iota(jnp.int32, (_CROWS, _LANES), 0) * _LANES
            + lax.broadcasted_iota(jnp.int32, (_CROWS, _LANES), 1))
    trash = jnp.int32(_ZFILL) + (gidx & jnp.int32(127))
    for b in range(_B):
        kept = jnp.concatenate(kept_bands[b], axis=0)
        lane_excl = lax.dot_general(kept, upper_strict,
                                    (((1,), (0,)), ((), ())),
                                    preferred_element_type=jnp.float32)
        rowtot = jnp.sum(kept, axis=1, keepdims=True)
        row_excl = lax.dot_general(lower_strict, rowtot,
                                   (((1,), (0,)), ((), ())),
                                   preferred_element_type=jnp.float32)
        rank = (row_excl + lane_excl).astype(jnp.int32)
        slot = jnp.where((kept > 0.5) & (rank < _ZFILL), rank, trash)
        sidx3_ref[b] = slot + b * jnp.int32(_AOUT)


def _sc_assemble(rows_hbm, sidx_hbm, out_hbm, idx_v, rows_v, zero_v, sem):
    c = lax.axis_index("c")
    s = lax.axis_index("s")
    for r in range(64):
        zero_v[r, :] = jnp.zeros((_RW,), jnp.float32)
    base = c * _AOUT + s * 64
    pltpu.sync_copy(zero_v, out_hbm.at[pl.ds(base, 64)])
    plsc.subcore_barrier()
    w = s * 2 + c
    pltpu.sync_copy(sidx_hbm.at[w], idx_v)
    pltpu.sync_copy(rows_hbm.at[w], rows_v)
    ch = idx_v.shape[0]
    cps = [pltpu.async_copy(rows_v.at[cc], out_hbm.at[idx_v.at[cc]], sem)
           for cc in range(ch)]
    for cp in cps:
        cp.wait()


def kernel(scores, deltas, anchors):
    B, N, _ = scores.shape
    fg = scores[:, :, 1]
    pad = _NPAD - N
    fg = jnp.pad(fg, ((0, 0), (0, pad)), constant_values=_NEG)
    fg = fg.reshape(B, _ROWS, _LANES)
    d = jnp.moveaxis(deltas, 2, 1)
    a = jnp.moveaxis(anchors, 2, 1)
    d = jnp.pad(d, ((0, 0), (0, 0), (0, pad)))
    a = jnp.pad(a, ((0, 0), (0, 0), (0, pad)))
    d = d.reshape(B, 4, _ROWS, _LANES)
    a = a.reshape(B, 4, _ROWS, _LANES)

    planes, sidx = pl.pallas_call(
        _prep_body,
        grid=(B,),
        in_specs=[
            pl.BlockSpec((1, _ROWS, _LANES), lambda b: (b, 0, 0)),
            pl.BlockSpec((1, 4, _ROWS, _LANES), lambda b: (b, 0, 0, 0)),
            pl.BlockSpec((1, 4, _ROWS, _LANES), lambda b: (b, 0, 0, 0)),
        ],
        out_specs=[
            pl.BlockSpec((1, 6, _ROWS, _LANES), lambda b: (b, 0, 0, 0)),
            pl.BlockSpec((1, _ROWS, _LANES), lambda b: (b, 0, 0)),
        ],
        out_shape=[
            jax.ShapeDtypeStruct((B, 6, _ROWS, _LANES), jnp.float32),
            jax.ShapeDtypeStruct((B, _ROWS, _LANES), jnp.int32),
        ],
    )(fg, d, a)

    def _sc_scatter(rows_flat, sidx_flat):
        nrow = rows_flat.shape[0]
        ch = nrow // (_NW * _LANES)
        rows3 = rows_flat.reshape(_NW, ch, _LANES, _RW)
        sidx3 = sidx_flat.reshape(_NW, ch, _LANES)
        return pl.kernel(
            _sc_compact,
            out_type=jax.ShapeDtypeStruct((B * _COUT, _RW), jnp.float32),
            mesh=plsc.VectorSubcoreMesh(core_axis_name="c",
                                        subcore_axis_name="s"),
            scratch_types=[
                pltpu.VMEM((ch, _LANES), jnp.int32),
                pltpu.VMEM((ch, _LANES, _RW), jnp.float32),
                pltpu.SemaphoreType.DMA,
            ],
            compiler_params=pltpu.CompilerParams(use_tc_tiling_on_sc=False),
        )(rows3, sidx3)

    rows = jnp.moveaxis(planes.reshape(B, 6, _NPAD), 1, 2)
    rows = jnp.pad(rows, ((0, 0), (0, 0), (0, _RW - 6)))
    compact = _sc_scatter(rows.reshape(B * _NPAD, _RW), sidx.reshape(-1))

    rows1 = compact.reshape(B, _COUT, _RW)
    sp = jnp.moveaxis(rows1[:, :_CN, 0:1], 1, 2)
    sp = sp.reshape(B, 1, _CROWS, _LANES)
    sidx2 = pl.pallas_call(
        _rank_body,
        grid=(B,),
        in_specs=[
            pl.BlockSpec((1, _COUT, _RW), lambda b: (b, 0, 0)),
            pl.BlockSpec((1, 1, _CROWS, _LANES), lambda b: (b, 0, 0, 0)),
        ],
        out_specs=pl.BlockSpec((1, _CN, 1), lambda b: (b, 0, 0)),
        out_shape=jax.ShapeDtypeStruct((B, _CN, 1), jnp.int32),
    )(rows1, sp)

    compact2 = _sc_scatter(rows1[:, :_CN].reshape(B * _CN, _RW),
                           sidx2.reshape(-1))

    rows2 = compact2.reshape(B, _COUT, _RW)
    cp2 = jnp.moveaxis(rows2, 1, 2)[:, 1:6, :_CN]
    cp2 = cp2.reshape(B, 5, _CROWS, _LANES)
    cp2 = jnp.pad(cp2, ((0, 0), (1, 0), (0, 0), (0, 0)))

    sidx3 = pl.pallas_call(
        _banded_body,
        in_specs=[
            pl.BlockSpec((B, 6, _CROWS, _LANES), lambda: (0, 0, 0, 0)),
            pl.BlockSpec((B, _COUT, _RW), lambda: (0, 0, 0)),
        ],
        out_specs=pl.BlockSpec((B, _CROWS, _LANES), lambda: (0, 0, 0)),
        out_shape=jax.ShapeDtypeStruct((B, _CROWS, _LANES), jnp.int32),
        scratch_shapes=[pltpu.VMEM((_B * _CN, 1), jnp.float32)],
    )(cp2, rows2)

    arows = rows2[:, :_CN].reshape(B, 16, 3, _LANES, _RW)
    arows = jnp.moveaxis(arows, 0, 1).reshape(_NW, 3, _LANES, _RW)
    aidx = sidx3.reshape(B, 16, 3, _LANES)
    aidx = jnp.moveaxis(aidx, 0, 1).reshape(_NW, 3, _LANES)
    out3 = pl.kernel(
        _sc_assemble,
        out_type=jax.ShapeDtypeStruct((B * _AOUT, _RW), jnp.float32),
        mesh=plsc.VectorSubcoreMesh(core_axis_name="c", subcore_axis_name="s"),
        scratch_types=[
            pltpu.VMEM((3, _LANES), jnp.int32),
            pltpu.VMEM((3, _LANES, _RW), jnp.float32),
            pltpu.VMEM((64, _RW), jnp.float32),
            pltpu.SemaphoreType.DMA,
        ],
        compiler_params=pltpu.CompilerParams(use_tc_tiling_on_sc=False),
    )(arows, aidx)
    return out3.reshape(B, _AOUT, _RW)[:, :_PROPOSALS, 1:5]
